# baseline (speedup 1.0000x reference)
import functools
import math

import jax
import jax.numpy as jnp
from jax import lax
from jax.experimental import pallas as pl
from jax.experimental.pallas import tpu as pltpu
from jax.experimental.pallas import tpu_sc as plsc

_VOCAB = 100000
_EMB = 300
_OUT = 128
_SCALE = math.sqrt(300.0)

_PROJ_ROW = 2000
_PROJ_NB = _VOCAB // _PROJ_ROW
_RING = 5


def _proj_body(t_any, w_v, b_v, o_any, *scr):
    tbuf = scr[0:_RING]
    obuf = scr[_RING:2 * _RING]
    tsem = scr[2 * _RING:3 * _RING]
    osem = scr[3 * _RING:4 * _RING]

    for s in range(_RING):
        pltpu.async_copy(
            t_any.at[pl.ds(s * _PROJ_ROW, _PROJ_ROW)], tbuf[s], tsem[s]
        )

    @pl.loop(0, _PROJ_NB, step=_RING)
    def main(g):
        for s in range(_RING):
            c = g + s
            pltpu.make_async_copy(
                t_any.at[pl.ds(0, _PROJ_ROW)], tbuf[s], tsem[s]
            ).wait()

            @pl.when(c >= _RING)
            def _wait_prev_write():
                pltpu.make_async_copy(
                    obuf[s], o_any.at[pl.ds(0, _PROJ_ROW)], osem[s]
                ).wait()

            acc = jnp.dot(tbuf[s][...], w_v[...], preferred_element_type=jnp.float32)
            obuf[s][...] = acc * _SCALE + b_v[...]
            pltpu.async_copy(
                obuf[s], o_any.at[pl.ds(c * _PROJ_ROW, _PROJ_ROW)], osem[s]
            )

            @pl.when(c + _RING < _PROJ_NB)
            def _refill():
                pltpu.async_copy(
                    t_any.at[pl.ds((c + _RING) * _PROJ_ROW, _PROJ_ROW)],
                    tbuf[s],
                    tsem[s],
                )

    for s in range(_RING):
        pltpu.make_async_copy(
            obuf[s], o_any.at[pl.ds(0, _PROJ_ROW)], osem[s]
        ).wait()


def _project_table(table, W, b):
    return pl.pallas_call(
        _proj_body,
        in_specs=[
            pl.BlockSpec(memory_space=pltpu.HBM),
            pl.BlockSpec(memory_space=pltpu.VMEM),
            pl.BlockSpec(memory_space=pltpu.VMEM),
        ],
        out_specs=pl.BlockSpec(memory_space=pltpu.HBM),
        out_shape=jax.ShapeDtypeStruct((_VOCAB, _OUT), jnp.float32),
        scratch_shapes=(
            [pltpu.VMEM((_PROJ_ROW, _EMB), jnp.float32)] * _RING
            + [pltpu.VMEM((_PROJ_ROW, _OUT), jnp.float32)] * _RING
            + [pltpu.SemaphoreType.DMA] * (2 * _RING)
        ),
    )(table, W, b.reshape(1, _OUT))


_NTOK = 4096 * 200
_NW = 32
_PER_W = _NTOK // _NW
_CHUNK = 200
_NCHUNK = _PER_W // _CHUNK
_NSLOT = 4


def _gather_kernel(tok_hbm, p_hbm, out_hbm, *scr):
    idx = scr[0:_NSLOT]
    rows = scr[_NSLOT:2 * _NSLOT]
    gsem = scr[2 * _NSLOT:3 * _NSLOT]
    wsem = scr[3 * _NSLOT:4 * _NSLOT]
    wid = lax.axis_index("s") * 2 + lax.axis_index("c")
    base = wid * _PER_W

    pltpu.sync_copy(tok_hbm.at[pl.ds(base, _CHUNK)], idx[0])
    pltpu.async_copy(p_hbm.at[idx[0]], rows[0], gsem[0])

    @pl.loop(0, _NCHUNK, step=_NSLOT)
    def main(g):
        for s in range(_NSLOT):
            c = g + s
            sn = (s + 1) % _NSLOT

            @pl.when(c + 1 < _NCHUNK)
            def _launch_next():
                @pl.when(c + 1 >= _NSLOT)
                def _drain_prev_write():
                    pltpu.make_async_copy(
                        rows[sn], out_hbm.at[pl.ds(base, _CHUNK)], wsem[sn]
                    ).wait()

                pltpu.sync_copy(
                    tok_hbm.at[pl.ds(base + (c + 1) * _CHUNK, _CHUNK)], idx[sn]
                )
                pltpu.async_copy(p_hbm.at[idx[sn]], rows[sn], gsem[sn])

            pltpu.make_async_copy(p_hbm.at[idx[s]], rows[s], gsem[s]).wait()
            pltpu.async_copy(
                rows[s], out_hbm.at[pl.ds(base + c * _CHUNK, _CHUNK)], wsem[s]
            )

    for s in range(_NSLOT):
        pltpu.make_async_copy(
            rows[s], out_hbm.at[pl.ds(base, _CHUNK)], wsem[s]
        ).wait()


@functools.partial(jax.jit, static_argnames=())
def kernel(tokens, table, W, b):
    proj = _project_table(table, W, b)
    tok_flat = tokens.reshape(_NTOK)

    sc_gather = pl.kernel(
        _gather_kernel,
        out_type=jax.ShapeDtypeStruct((_NTOK, _OUT), jnp.float32),
        mesh=plsc.VectorSubcoreMesh(core_axis_name="c", subcore_axis_name="s"),
        scratch_types=(
            [pltpu.VMEM((_CHUNK,), jnp.int32)] * _NSLOT
            + [pltpu.VMEM((_CHUNK, _OUT), jnp.float32)] * _NSLOT
            + [pltpu.SemaphoreType.DMA] * (2 * _NSLOT)
        ),
    )
    out_flat = sc_gather(tok_flat, proj)
    return out_flat.reshape(4096, 200, _OUT)
